# baseline (speedup 1.0000x reference)
import functools

import jax
import jax.numpy as jnp
from jax import lax
from jax.experimental import pallas as pl
from jax.experimental.pallas import tpu as pltpu
from jax.experimental.pallas import tpu_sc as plsc

F = 16
NCORE = 2
NSUB = 16
NW = NCORE * NSUB
CH = 128
KJ = 8
GRP = KJ * CH

N = 50000
E = 800000
G = -(-(E // NW) // GRP)
G = G + (G % 2)
E_PAD = NW * G * GRP
N_TAB = 50176
PER_SUB = N_TAB // NSUB
ZB = 392
DUMMY = N

BN = 1024
BE = 4096

_f32 = jnp.float32
_i32 = jnp.int32


def _relu(x):
    return jnp.maximum(x, 0.0)


def _onehot(tf_ref, rows):
    return (tf_ref[...].astype(_i32)
            == lax.broadcasted_iota(_i32, (rows, 8), 1)).astype(_f32)


def _tc_init_nodes(ntf, T0):
    def body(ntf_ref, T0_ref, a_ref, c_ref, u_ref):
        acu = jnp.dot(_onehot(ntf_ref, BN), T0_ref[...],
                      preferred_element_type=_f32)
        a_ref[...] = acu[:, 0:16]
        c_ref[...] = acu[:, 16:32]
        u_ref[...] = acu[:, 32:48]

    o = jax.ShapeDtypeStruct((N_TAB, F), _f32)
    return pl.pallas_call(
        body,
        grid=(N_TAB // BN,),
        in_specs=[pl.BlockSpec((BN, 1), lambda i: (i, 0)),
                  pl.BlockSpec((8, 48), lambda i: (0, 0))],
        out_specs=[pl.BlockSpec((BN, F), lambda i: (i, 0))] * 3,
        out_shape=[o, o, o],
    )(ntf, T0)


def _tc_init_edges(etf, Tb0):
    def body(etf_ref, Tb0_ref, eb_ref):
        eb_ref[...] = jnp.dot(_onehot(etf_ref, BE), Tb0_ref[...],
                              preferred_element_type=_f32)

    return pl.pallas_call(
        body,
        grid=(E_PAD // BE,),
        in_specs=[pl.BlockSpec((BE, 1), lambda i: (i, 0)),
                  pl.BlockSpec((8, F), lambda i: (0, 0))],
        out_specs=pl.BlockSpec((BE, F), lambda i: (i, 0)),
        out_shape=jax.ShapeDtypeStruct((E_PAD, F), _f32),
    )(etf, Tb0)


def _tc_edge(er1, etf, ceW2, ceb2, M2, Wb2, Tb):
    def body(er1_ref, etf_ref, W2_ref, b2_ref, M2_ref, Wb2_ref, Tb_ref,
             er_ref, w_ref, ebn_ref):
        er = _relu(jnp.dot(er1_ref[...], W2_ref[...],
                           preferred_element_type=_f32) + b2_ref[...])
        er_ref[...] = er
        w_ref[...] = jnp.dot(er, M2_ref[...], preferred_element_type=_f32)
        ebn_ref[...] = (jnp.dot(_onehot(etf_ref, BE), Tb_ref[...],
                                preferred_element_type=_f32)
                        + jnp.dot(er, Wb2_ref[...],
                                  preferred_element_type=_f32))

    full = lambda shape: pl.BlockSpec(shape, lambda i: (0, 0))
    o = jax.ShapeDtypeStruct((E_PAD, F), _f32)
    return pl.pallas_call(
        body,
        grid=(E_PAD // BE,),
        in_specs=[pl.BlockSpec((BE, F), lambda i: (i, 0)),
                  pl.BlockSpec((BE, 1), lambda i: (i, 0)),
                  full((F, F)), full((1, F)), full((F, F)), full((F, F)),
                  full((8, F))],
        out_specs=[pl.BlockSpec((BE, F), lambda i: (i, 0))] * 3,
        out_shape=[o, o, o],
    )(er1, etf, ceW2, ceb2, M2, Wb2, Tb)


def _tc_node(S0, S1, ntf, cnb1, cnW2, cnb2, TN, W3):
    def body(s0_ref, s1_ref, ntf_ref, b1_ref, W2_ref, b2_ref, TN_ref, W3_ref,
             xn_ref, a_ref, c_ref, u_ref):
        xn1 = _relu(s0_ref[...] + s1_ref[...] + b1_ref[...])
        xn = _relu(jnp.dot(xn1, W2_ref[...], preferred_element_type=_f32)
                   + b2_ref[...])
        xn_ref[...] = xn
        acu = (jnp.dot(_onehot(ntf_ref, BN), TN_ref[...],
                       preferred_element_type=_f32)
               + jnp.dot(xn, W3_ref[...], preferred_element_type=_f32))
        a_ref[...] = acu[:, 0:16]
        c_ref[...] = acu[:, 16:32]
        u_ref[...] = acu[:, 32:48]

    full = lambda shape: pl.BlockSpec(shape, lambda i: (0, 0))
    o = jax.ShapeDtypeStruct((N_TAB, F), _f32)
    return pl.pallas_call(
        body,
        grid=(N_TAB // BN,),
        in_specs=[pl.BlockSpec((BN, F), lambda i: (i, 0)),
                  pl.BlockSpec((BN, F), lambda i: (i, 0)),
                  pl.BlockSpec((BN, 1), lambda i: (i, 0)),
                  full((1, F)), full((F, F)), full((1, F)),
                  full((8, 48)), full((F, 48))],
        out_specs=[pl.BlockSpec((BN, F), lambda i: (i, 0))] * 4,
        out_shape=[o, o, o, o],
    )(S0, S1, ntf, cnb1, cnW2, cnb2, TN, W3)


def _tc_dec(x, W1, b1, W2, b2, W3p, b3p, rows, block):
    def body(x_ref, W1_ref, b1_ref, W2_ref, b2_ref, W3_ref, b3_ref, o_ref):
        h = _relu(jnp.dot(x_ref[...], W1_ref[...],
                          preferred_element_type=_f32) + b1_ref[...])
        h = _relu(jnp.dot(h, W2_ref[...], preferred_element_type=_f32)
                  + b2_ref[...])
        o_ref[...] = jnp.dot(h, W3_ref[...],
                             preferred_element_type=_f32) + b3_ref[...]

    full = lambda shape: pl.BlockSpec(shape, lambda i: (0, 0))
    return pl.pallas_call(
        body,
        grid=(rows // block,),
        in_specs=[pl.BlockSpec((block, F), lambda i: (i, 0)),
                  full((F, F)), full((1, F)), full((F, F)), full((1, F)),
                  full((F, 8)), full((1, 8))],
        out_specs=pl.BlockSpec((block, 8), lambda i: (i, 0)),
        out_shape=jax.ShapeDtypeStruct((rows, 8), _f32),
    )(x, W1, b1, W2, b2, W3p, b3p)


@functools.lru_cache(maxsize=None)
def _sc_gather_kernel():
    mesh = plsc.VectorSubcoreMesh(core_axis_name="c", subcore_axis_name="s")
    buf = [pltpu.VMEM((KJ, CH), _i32),
           pltpu.VMEM((KJ, CH), _i32),
           pltpu.VMEM((GRP, F), _f32),
           pltpu.VMEM((GRP, F), _f32),
           pltpu.VMEM((GRP, F), _f32),
           pltpu.SemaphoreType.DMA,
           pltpu.SemaphoreType.DMA,
           pltpu.SemaphoreType.DMA,
           pltpu.SemaphoreType.DMA]
    return functools.partial(
        pl.kernel, mesh=mesh,
        compiler_params=pltpu.CompilerParams(use_tc_tiling_on_sc=False),
        out_type=jax.ShapeDtypeStruct((E_PAD, F), _f32),
        scratch_types=buf + buf)(_sc_gather_body)


def _sc_gather_body(a_hbm, c_hbm, eb_hbm, src2_hbm, dst2_hbm, out_hbm, *s):
    idxs = (s[0], s[9])
    idxd = (s[1], s[10])
    rowsA = (s[2], s[11])
    rowsC = (s[3], s[12])
    ebv = (s[4], s[13])
    semI = (s[5], s[14])
    semG = (s[6], s[15])
    semE = (s[7], s[16])
    semO = (s[8], s[17])
    wid = lax.axis_index("s") * NCORE + lax.axis_index("c")
    base_g = wid * G

    def fire_idx(g, b):
        row128 = (base_g + g) * KJ
        pltpu.async_copy(src2_hbm.at[pl.ds(row128, KJ), :], idxs[b], semI[b])
        pltpu.async_copy(dst2_hbm.at[pl.ds(row128, KJ), :], idxd[b], semI[b])

    def wait_idx(b):
        pltpu.make_async_copy(src2_hbm.at[pl.ds(0, KJ), :], idxs[b],
                              semI[b]).wait()
        pltpu.make_async_copy(dst2_hbm.at[pl.ds(0, KJ), :], idxd[b],
                              semI[b]).wait()

    def fire_grp(g, b):
        off = (base_g + g) * GRP
        pltpu.async_copy(eb_hbm.at[pl.ds(off, GRP), :], ebv[b], semE[b])
        for j in range(KJ):
            pltpu.async_copy(a_hbm.at[idxs[b].at[j]],
                             rowsA[b].at[pl.ds(j * CH, CH), :], semG[b])
            pltpu.async_copy(c_hbm.at[idxd[b].at[j]],
                             rowsC[b].at[pl.ds(j * CH, CH), :], semG[b])

    def wait_grp(b):
        pltpu.make_async_copy(eb_hbm.at[pl.ds(0, GRP), :], ebv[b],
                              semE[b]).wait()
        pltpu.make_async_copy(a_hbm.at[pl.ds(0, GRP), :], rowsA[b],
                              semG[b]).wait()
        pltpu.make_async_copy(a_hbm.at[pl.ds(0, GRP), :], rowsC[b],
                              semG[b]).wait()

    def compute_store(g, b):
        ra, rc, eb_ = rowsA[b], rowsC[b], ebv[b]

        @plsc.parallel_loop(0, GRP, step=1, unroll=8)
        def _(i):
            ra[i] = jnp.maximum(ra[i] + rc[i] + eb_[i], 0.0)

        off = (base_g + g) * GRP
        pltpu.async_copy(ra, out_hbm.at[pl.ds(off, GRP), :], semO[b])

    def wait_store(b):
        pltpu.make_async_copy(rowsA[b], out_hbm.at[pl.ds(0, GRP), :],
                              semO[b]).wait()

    fire_idx(0, 0)
    wait_idx(0)
    fire_grp(0, 0)
    fire_idx(1, 1)

    def pair(k, _):
        g0 = 2 * k
        wait_idx(1)
        fire_grp(g0 + 1, 1)
        wait_grp(0)
        compute_store(g0, 0)
        wait_grp(1)
        compute_store(g0 + 1, 1)

        @pl.when(k < G // 2 - 1)
        def _():
            wait_store(0)
            fire_idx(g0 + 2, 0)
            wait_idx(0)
            fire_grp(g0 + 2, 0)
            wait_store(1)
            fire_idx(g0 + 3, 1)

        return 0

    lax.fori_loop(0, G // 2, pair, 0)
    wait_store(0)
    wait_store(1)


@functools.lru_cache(maxsize=None)
def _sc_scatter_kernel():
    mesh = plsc.VectorSubcoreMesh(core_axis_name="c", subcore_axis_name="s")
    buf = [pltpu.VMEM((KJ, CH), _i32),
           pltpu.VMEM((KJ, CH), _i32),
           pltpu.VMEM((GRP, F), _f32),
           pltpu.VMEM((GRP, F), _f32),
           pltpu.SemaphoreType.DMA,
           pltpu.SemaphoreType.DMA,
           pltpu.SemaphoreType.DMA,
           pltpu.SemaphoreType.DMA]
    return functools.partial(
        pl.kernel, mesh=mesh,
        compiler_params=pltpu.CompilerParams(use_tc_tiling_on_sc=False),
        out_type=jax.ShapeDtypeStruct((NCORE, N_TAB, F), _f32),
        scratch_types=buf + buf + [
            pltpu.VMEM((ZB, F), _f32),
            pltpu.VMEM_SHARED((N_TAB, F), _f32),
        ])(_sc_scatter_body)


def _sc_scatter_body(u_hbm, w_hbm, src2_hbm, dst2_hbm, out_hbm, *s):
    idxs = (s[0], s[8])
    idxd = (s[1], s[9])
    rowsU = (s[2], s[10])
    wv = (s[3], s[11])
    semI = (s[4], s[12])
    semG = (s[5], s[13])
    semW = (s[6], s[14])
    semS = (s[7], s[15])
    zbuf = s[16]
    acc = s[17]
    cid = lax.axis_index("c")
    sid = lax.axis_index("s")
    wid = sid * NCORE + cid
    base_g = wid * G

    @plsc.parallel_loop(0, ZB, step=1, unroll=8)
    def _(i):
        zbuf[i] = jnp.zeros((F,), _f32)

    for r in range(PER_SUB // ZB):
        pltpu.sync_copy(zbuf, acc.at[pl.ds(sid * PER_SUB + r * ZB, ZB), :])
    plsc.subcore_barrier()

    def fire_idx(g, b):
        row128 = (base_g + g) * KJ
        pltpu.async_copy(src2_hbm.at[pl.ds(row128, KJ), :], idxs[b], semI[b])
        pltpu.async_copy(dst2_hbm.at[pl.ds(row128, KJ), :], idxd[b], semI[b])

    def wait_idx(b):
        pltpu.make_async_copy(src2_hbm.at[pl.ds(0, KJ), :], idxs[b],
                              semI[b]).wait()
        pltpu.make_async_copy(dst2_hbm.at[pl.ds(0, KJ), :], idxd[b],
                              semI[b]).wait()

    def fire_grp(g, b):
        off = (base_g + g) * GRP
        pltpu.async_copy(w_hbm.at[pl.ds(off, GRP), :], wv[b], semW[b])
        for j in range(KJ):
            pltpu.async_copy(u_hbm.at[idxs[b].at[j]],
                             rowsU[b].at[pl.ds(j * CH, CH), :], semG[b])

    def wait_grp(b):
        pltpu.make_async_copy(w_hbm.at[pl.ds(0, GRP), :], wv[b],
                              semW[b]).wait()
        pltpu.make_async_copy(u_hbm.at[pl.ds(0, GRP), :], rowsU[b],
                              semG[b]).wait()

    def compute_scatter(b):
        ru, w_ = rowsU[b], wv[b]

        @plsc.parallel_loop(0, GRP, step=1, unroll=8)
        def _(i):
            ru[i] = ru[i] + w_[i]

        for j in range(KJ):
            pltpu.async_copy(ru.at[pl.ds(j * CH, CH), :],
                             acc.at[idxd[b].at[j]], semS[b], add=True)

    def wait_scat(b):
        pltpu.make_async_copy(u_hbm.at[pl.ds(0, GRP), :], rowsU[b],
                              semS[b]).wait()

    fire_idx(0, 0)
    wait_idx(0)
    fire_grp(0, 0)
    fire_idx(1, 1)

    def pair(k, _):
        g0 = 2 * k
        wait_idx(1)
        fire_grp(g0 + 1, 1)
        wait_grp(0)
        compute_scatter(0)
        wait_grp(1)
        compute_scatter(1)

        @pl.when(k < G // 2 - 1)
        def _():
            wait_scat(0)
            fire_idx(g0 + 2, 0)
            wait_idx(0)
            fire_grp(g0 + 2, 0)
            wait_scat(1)
            fire_idx(g0 + 3, 1)

        return 0

    lax.fori_loop(0, G // 2, pair, 0)
    wait_scat(0)
    wait_scat(1)
    plsc.subcore_barrier()

    for r in range(PER_SUB // ZB):
        pltpu.sync_copy(acc.at[pl.ds(sid * PER_SUB + r * ZB, ZB), :], zbuf)
        pltpu.sync_copy(zbuf,
                        out_hbm.at[cid, pl.ds(sid * PER_SUB + r * ZB, ZB), :])


def kernel(nte, ete, neW1, neb1, neW2, neb2, eeW1, eeb1, eeW2, eeb2,
           ceW1, ceb1, ceW2, ceb2, cnW1, cnb1, cnW2, cnb2,
           ndW1, ndb1, ndW2, ndb2, ndW3, ndb3,
           edW1, edb1, edW2, edb2, edW3, edb3,
           x_node_types, x_edge_types, edge_index, steps):
    relu = _relu

    def mlp2(x, W1, b1, W2, b2):
        return relu(relu(x @ W1 + b1) @ W2 + b2)

    ntab = mlp2(nte, neW1, neb1, neW2, neb2)
    etab = mlp2(ete, eeW1, eeb1, eeW2, eeb2)
    Wa1, Wa2 = ceW1[0:16], ceW1[16:32]
    Wb1, Wb2 = ceW1[32:48], ceW1[48:64]
    Wc1, Wc2 = ceW1[64:80], ceW1[80:96]
    Wu1, Wu2 = cnW1[0:16], cnW1[16:32]
    M2 = cnW1[32:48]

    def pad8(t):
        return jnp.pad(t, ((0, 8 - t.shape[0]), (0, 0)))

    TN = pad8(jnp.concatenate([ntab @ Wa1, ntab @ Wc1, ntab @ Wu1], axis=1))
    T0 = pad8(jnp.concatenate([ntab @ (Wa1 + Wa2), ntab @ (Wc1 + Wc2),
                               ntab @ (Wu1 + Wu2)], axis=1))
    W3 = jnp.concatenate([Wa2, Wc2, Wu2], axis=1)
    Tb = pad8(etab @ Wb1 + ceb1)
    Tb0 = pad8(etab @ (Wb1 + Wb2) + ceb1)

    row = lambda b: b.reshape(1, F)
    ceb2r, cnb1r, cnb2r = row(ceb2), row(cnb1), row(cnb2)
    ndb1r, ndb2r, edb1r, edb2r = row(ndb1), row(ndb2), row(edb1), row(edb2)
    padW3 = lambda W: jnp.pad(W, ((0, 0), (0, 8 - W.shape[1])))
    ndW3p, edW3p = padW3(ndW3), padW3(edW3)
    ndb3p = jnp.pad(ndb3, (0, 8 - ndb3.shape[0])).reshape(1, 8)
    edb3p = jnp.pad(edb3, (0, 8 - edb3.shape[0])).reshape(1, 8)

    src = edge_index[0].astype(_i32)
    dst = edge_index[1].astype(_i32)
    padE = E_PAD - E
    src2 = jnp.concatenate([src, jnp.full((padE,), DUMMY, _i32)]
                           ).reshape(E_PAD // CH, CH)
    dst2 = jnp.concatenate(
        [dst, DUMMY + (jnp.arange(padE, dtype=_i32) % (N_TAB - N))]
    ).reshape(E_PAD // CH, CH)
    ntf = jnp.pad(x_node_types.astype(_f32), (0, N_TAB - N)
                  ).reshape(N_TAB, 1)
    etf = jnp.pad(x_edge_types.astype(_f32), (0, padE)).reshape(E_PAD, 1)

    a0, c0, u0 = _tc_init_nodes(ntf, T0)
    eb0 = _tc_init_edges(etf, Tb0)
    er0 = jnp.zeros((E_PAD, F), _f32)
    xn0 = jnp.zeros((N_TAB, F), _f32)

    def body(t, carry):
        a, c, u, eb, er, xn = carry
        er1 = _sc_gather_kernel()(a, c, eb, src2, dst2)
        er, w, ebn = _tc_edge(er1, etf, ceW2, ceb2r, M2, Wb2, Tb)
        S2 = _sc_scatter_kernel()(u, w, src2, dst2)
        xn, a, c, u = _tc_node(S2[0], S2[1], ntf, cnb1r, cnW2, cnb2r, TN, W3)
        return (a, c, u, ebn, er, xn)

    a, c, u, eb, er, xn = lax.fori_loop(
        0, steps, body, (a0, c0, u0, eb0, er0, xn0))

    pn = _tc_dec(xn, ndW1, ndb1r, ndW2, ndb2r, ndW3p, ndb3p, N_TAB, BN)
    pe = _tc_dec(er, edW1, edb1r, edW2, edb2r, edW3p, edb3p, E_PAD, BE)
    return (pn[:N, :3], pe[:E, :3])


# speedup vs baseline: 1.8797x; 1.8797x over previous
import functools

import jax
import jax.numpy as jnp
from jax import lax
from jax.experimental import pallas as pl
from jax.experimental.pallas import tpu as pltpu
from jax.experimental.pallas import tpu_sc as plsc

F = 16
NCORE = 2
NSUB = 16
NW = NCORE * NSUB
CH = 128
KJ = 8
GRP = KJ * CH

N = 50000
E = 800000
G = -(-(E // NW) // GRP)
G = G + (G % 2)
E_PAD = NW * G * GRP
EP8 = E_PAD // 8
GRPR = GRP // 8
N_TAB = 50176
PER_SUB = N_TAB // NSUB
ZB = 392
DUMMY = N

BN = 1024
BRE = 512

_f32 = jnp.float32
_i32 = jnp.int32


def _relu(x):
    return jnp.maximum(x, 0.0)


def _onehot(tf_ref, rows):
    return (tf_ref[...].astype(_i32)
            == lax.broadcasted_iota(_i32, (rows, 8), 1)).astype(_f32)


def _tc_init_nodes(ntf, T0):
    def body(ntf_ref, T0_ref, a_ref, c_ref, u_ref):
        acu = jnp.dot(_onehot(ntf_ref, BN), T0_ref[...],
                      preferred_element_type=_f32)
        a_ref[...] = acu[:, 0:16]
        c_ref[...] = acu[:, 16:32]
        u_ref[...] = acu[:, 32:48]

    o = jax.ShapeDtypeStruct((N_TAB, F), _f32)
    return pl.pallas_call(
        body,
        grid=(N_TAB // BN,),
        in_specs=[pl.BlockSpec((BN, 1), lambda i: (i, 0)),
                  pl.BlockSpec((8, 48), lambda i: (0, 0))],
        out_specs=[pl.BlockSpec((BN, F), lambda i: (i, 0))] * 3,
        out_shape=[o, o, o],
    )(ntf, T0)


def _type_mix(et, T_ref):
    return (jnp.where(et == 0.0, T_ref[0:1, :], 0.0)
            + jnp.where(et == 1.0, T_ref[1:2, :], 0.0)
            + jnp.where(et == 2.0, T_ref[2:3, :], 0.0))


def _tc_init_edges(etr, Tb0t):
    def body(et_ref, Tb0_ref, eb_ref):
        eb_ref[...] = _type_mix(et_ref[...], Tb0_ref)

    return pl.pallas_call(
        body,
        grid=(EP8 // BRE,),
        in_specs=[pl.BlockSpec((BRE, 128), lambda i: (i, 0)),
                  pl.BlockSpec((8, 128), lambda i: (0, 0))],
        out_specs=pl.BlockSpec((BRE, 128), lambda i: (i, 0)),
        out_shape=jax.ShapeDtypeStruct((EP8, 128), _f32),
    )(etr, Tb0t)


def _tc_edge(er1p, etr, W2d, b2t, M2d, Wb2d, Tbt):
    def body(er1_ref, et_ref, W2_ref, b2_ref, M2_ref, Wb2_ref, Tb_ref,
             er_ref, w_ref, ebn_ref):
        er = _relu(jnp.dot(er1_ref[...], W2_ref[...],
                           preferred_element_type=_f32) + b2_ref[...])
        er_ref[...] = er
        w_ref[...] = jnp.dot(er, M2_ref[...], preferred_element_type=_f32)
        ebn_ref[...] = (_type_mix(et_ref[...], Tb_ref)
                        + jnp.dot(er, Wb2_ref[...],
                                  preferred_element_type=_f32))

    full = lambda shape: pl.BlockSpec(shape, lambda i: (0, 0))
    o = jax.ShapeDtypeStruct((EP8, 128), _f32)
    return pl.pallas_call(
        body,
        grid=(EP8 // BRE,),
        in_specs=[pl.BlockSpec((BRE, 128), lambda i: (i, 0)),
                  pl.BlockSpec((BRE, 128), lambda i: (i, 0)),
                  full((128, 128)), full((1, 128)), full((128, 128)),
                  full((128, 128)), full((8, 128))],
        out_specs=[pl.BlockSpec((BRE, 128), lambda i: (i, 0))] * 3,
        out_shape=[o, o, o],
    )(er1p, etr, W2d, b2t, M2d, Wb2d, Tbt)


def _tc_node(S0, S1, ntf, cnb1, cnW2, cnb2, TN, W3):
    def body(s0_ref, s1_ref, ntf_ref, b1_ref, W2_ref, b2_ref, TN_ref, W3_ref,
             xn_ref, a_ref, c_ref, u_ref):
        xn1 = _relu(s0_ref[...] + s1_ref[...] + b1_ref[...])
        xn = _relu(jnp.dot(xn1, W2_ref[...], preferred_element_type=_f32)
                   + b2_ref[...])
        xn_ref[...] = xn
        acu = (jnp.dot(_onehot(ntf_ref, BN), TN_ref[...],
                       preferred_element_type=_f32)
               + jnp.dot(xn, W3_ref[...], preferred_element_type=_f32))
        a_ref[...] = acu[:, 0:16]
        c_ref[...] = acu[:, 16:32]
        u_ref[...] = acu[:, 32:48]

    full = lambda shape: pl.BlockSpec(shape, lambda i: (0, 0))
    o = jax.ShapeDtypeStruct((N_TAB, F), _f32)
    return pl.pallas_call(
        body,
        grid=(N_TAB // BN,),
        in_specs=[pl.BlockSpec((BN, F), lambda i: (i, 0)),
                  pl.BlockSpec((BN, F), lambda i: (i, 0)),
                  pl.BlockSpec((BN, 1), lambda i: (i, 0)),
                  full((1, F)), full((F, F)), full((1, F)),
                  full((8, 48)), full((F, 48))],
        out_specs=[pl.BlockSpec((BN, F), lambda i: (i, 0))] * 4,
        out_shape=[o, o, o, o],
    )(S0, S1, ntf, cnb1, cnW2, cnb2, TN, W3)


def _tc_dec(x, W1, b1, W2, b2, W3p, b3p, rows, block):
    def body(x_ref, W1_ref, b1_ref, W2_ref, b2_ref, W3_ref, b3_ref, o_ref):
        h = _relu(jnp.dot(x_ref[...], W1_ref[...],
                          preferred_element_type=_f32) + b1_ref[...])
        h = _relu(jnp.dot(h, W2_ref[...], preferred_element_type=_f32)
                  + b2_ref[...])
        o_ref[...] = jnp.dot(h, W3_ref[...],
                             preferred_element_type=_f32) + b3_ref[...]

    full = lambda shape: pl.BlockSpec(shape, lambda i: (0, 0))
    return pl.pallas_call(
        body,
        grid=(rows // block,),
        in_specs=[pl.BlockSpec((block, F), lambda i: (i, 0)),
                  full((F, F)), full((1, F)), full((F, F)), full((1, F)),
                  full((F, 8)), full((1, 8))],
        out_specs=pl.BlockSpec((block, 8), lambda i: (i, 0)),
        out_shape=jax.ShapeDtypeStruct((rows, 8), _f32),
    )(x, W1, b1, W2, b2, W3p, b3p)


def _tc_dec_edge(xp, W1d, b1t, W2d, b2t, W3d, b3t):
    def body(x_ref, W1_ref, b1_ref, W2_ref, b2_ref, W3_ref, b3_ref, o_ref):
        h = _relu(jnp.dot(x_ref[...], W1_ref[...],
                          preferred_element_type=_f32) + b1_ref[...])
        h = _relu(jnp.dot(h, W2_ref[...], preferred_element_type=_f32)
                  + b2_ref[...])
        o_ref[...] = jnp.dot(h, W3_ref[...],
                             preferred_element_type=_f32) + b3_ref[...]

    full = lambda shape: pl.BlockSpec(shape, lambda i: (0, 0))
    return pl.pallas_call(
        body,
        grid=(EP8 // BRE,),
        in_specs=[pl.BlockSpec((BRE, 128), lambda i: (i, 0)),
                  full((128, 128)), full((1, 128)), full((128, 128)),
                  full((1, 128)), full((128, 64)), full((1, 64))],
        out_specs=pl.BlockSpec((BRE, 64), lambda i: (i, 0)),
        out_shape=jax.ShapeDtypeStruct((EP8, 64), _f32),
    )(xp, W1d, b1t, W2d, b2t, W3d, b3t)


@functools.lru_cache(maxsize=None)
def _sc_gather_kernel():
    mesh = plsc.VectorSubcoreMesh(core_axis_name="c", subcore_axis_name="s")
    buf = [pltpu.VMEM((KJ, CH), _i32),
           pltpu.VMEM((KJ, CH), _i32),
           pltpu.VMEM((GRP, F), _f32),
           pltpu.VMEM((GRP, F), _f32),
           pltpu.VMEM((GRPR, 128), _f32),
           pltpu.SemaphoreType.DMA,
           pltpu.SemaphoreType.DMA,
           pltpu.SemaphoreType.DMA,
           pltpu.SemaphoreType.DMA]
    return functools.partial(
        pl.kernel, mesh=mesh,
        compiler_params=pltpu.CompilerParams(use_tc_tiling_on_sc=False),
        out_type=jax.ShapeDtypeStruct((EP8, 128), _f32),
        scratch_types=buf + buf)(_sc_gather_body)


def _sc_gather_body(a_hbm, c_hbm, eb_hbm, src2_hbm, dst2_hbm, out_hbm, *s):
    idxs = (s[0], s[9])
    idxd = (s[1], s[10])
    rowsA = (s[2], s[11])
    rowsC = (s[3], s[12])
    ebv = (s[4], s[13])
    semI = (s[5], s[14])
    semG = (s[6], s[15])
    semE = (s[7], s[16])
    semO = (s[8], s[17])
    wid = lax.axis_index("s") * NCORE + lax.axis_index("c")
    base_g = wid * G

    def fire_idx(g, b):
        row128 = (base_g + g) * KJ
        pltpu.async_copy(src2_hbm.at[pl.ds(row128, KJ), :], idxs[b], semI[b])
        pltpu.async_copy(dst2_hbm.at[pl.ds(row128, KJ), :], idxd[b], semI[b])

    def wait_idx(b):
        pltpu.make_async_copy(src2_hbm.at[pl.ds(0, KJ), :], idxs[b],
                              semI[b]).wait()
        pltpu.make_async_copy(dst2_hbm.at[pl.ds(0, KJ), :], idxd[b],
                              semI[b]).wait()

    def fire_grp(g, b):
        offp = (base_g + g) * GRPR
        pltpu.async_copy(eb_hbm.at[pl.ds(offp, GRPR), :], ebv[b], semE[b])
        for j in range(KJ):
            pltpu.async_copy(a_hbm.at[idxs[b].at[j]],
                             rowsA[b].at[pl.ds(j * CH, CH), :], semG[b])
            pltpu.async_copy(c_hbm.at[idxd[b].at[j]],
                             rowsC[b].at[pl.ds(j * CH, CH), :], semG[b])

    def wait_grp(b):
        pltpu.make_async_copy(eb_hbm.at[pl.ds(0, GRPR), :], ebv[b],
                              semE[b]).wait()
        pltpu.make_async_copy(a_hbm.at[pl.ds(0, GRP), :], rowsA[b],
                              semG[b]).wait()
        pltpu.make_async_copy(a_hbm.at[pl.ds(0, GRP), :], rowsC[b],
                              semG[b]).wait()

    def compute_store(g, b):
        ra, rc, eb_ = rowsA[b], rowsC[b], ebv[b]

        @plsc.parallel_loop(0, GRPR, step=1, unroll=2)
        def _(i):
            for j in range(8):
                sl = pl.ds(j * F, F)
                eb_[i, sl] = jnp.maximum(
                    ra[8 * i + j] + rc[8 * i + j] + eb_[i, sl], 0.0)

        offp = (base_g + g) * GRPR
        pltpu.async_copy(eb_, out_hbm.at[pl.ds(offp, GRPR), :], semO[b])

    def wait_store(b):
        pltpu.make_async_copy(ebv[b], out_hbm.at[pl.ds(0, GRPR), :],
                              semO[b]).wait()

    fire_idx(0, 0)
    wait_idx(0)
    fire_grp(0, 0)
    fire_idx(1, 1)

    def pair(k, _):
        g0 = 2 * k
        wait_idx(1)
        fire_grp(g0 + 1, 1)
        wait_grp(0)
        compute_store(g0, 0)
        wait_grp(1)
        compute_store(g0 + 1, 1)

        @pl.when(k < G // 2 - 1)
        def _():
            wait_store(0)
            fire_idx(g0 + 2, 0)
            wait_idx(0)
            fire_grp(g0 + 2, 0)
            wait_store(1)
            fire_idx(g0 + 3, 1)

        return 0

    lax.fori_loop(0, G // 2, pair, 0)
    wait_store(0)
    wait_store(1)


@functools.lru_cache(maxsize=None)
def _sc_scatter_kernel():
    mesh = plsc.VectorSubcoreMesh(core_axis_name="c", subcore_axis_name="s")
    buf = [pltpu.VMEM((KJ, CH), _i32),
           pltpu.VMEM((KJ, CH), _i32),
           pltpu.VMEM((GRP, F), _f32),
           pltpu.VMEM((GRPR, 128), _f32),
           pltpu.SemaphoreType.DMA,
           pltpu.SemaphoreType.DMA,
           pltpu.SemaphoreType.DMA,
           pltpu.SemaphoreType.DMA]
    return functools.partial(
        pl.kernel, mesh=mesh,
        compiler_params=pltpu.CompilerParams(use_tc_tiling_on_sc=False),
        out_type=jax.ShapeDtypeStruct((NCORE, N_TAB, F), _f32),
        scratch_types=buf + buf + [
            pltpu.VMEM((ZB, F), _f32),
            pltpu.VMEM_SHARED((N_TAB, F), _f32),
        ])(_sc_scatter_body)


def _sc_scatter_body(u_hbm, w_hbm, src2_hbm, dst2_hbm, out_hbm, *s):
    idxs = (s[0], s[8])
    idxd = (s[1], s[9])
    rowsU = (s[2], s[10])
    wv = (s[3], s[11])
    semI = (s[4], s[12])
    semG = (s[5], s[13])
    semW = (s[6], s[14])
    semS = (s[7], s[15])
    zbuf = s[16]
    acc = s[17]
    cid = lax.axis_index("c")
    sid = lax.axis_index("s")
    wid = sid * NCORE + cid
    base_g = wid * G

    @plsc.parallel_loop(0, ZB, step=1, unroll=8)
    def _(i):
        zbuf[i] = jnp.zeros((F,), _f32)

    for r in range(PER_SUB // ZB):
        pltpu.sync_copy(zbuf, acc.at[pl.ds(sid * PER_SUB + r * ZB, ZB), :])
    plsc.subcore_barrier()

    def fire_idx(g, b):
        row128 = (base_g + g) * KJ
        pltpu.async_copy(src2_hbm.at[pl.ds(row128, KJ), :], idxs[b], semI[b])
        pltpu.async_copy(dst2_hbm.at[pl.ds(row128, KJ), :], idxd[b], semI[b])

    def wait_idx(b):
        pltpu.make_async_copy(src2_hbm.at[pl.ds(0, KJ), :], idxs[b],
                              semI[b]).wait()
        pltpu.make_async_copy(dst2_hbm.at[pl.ds(0, KJ), :], idxd[b],
                              semI[b]).wait()

    def fire_grp(g, b):
        offp = (base_g + g) * GRPR
        pltpu.async_copy(w_hbm.at[pl.ds(offp, GRPR), :], wv[b], semW[b])
        for j in range(KJ):
            pltpu.async_copy(u_hbm.at[idxs[b].at[j]],
                             rowsU[b].at[pl.ds(j * CH, CH), :], semG[b])

    def wait_grp(b):
        pltpu.make_async_copy(w_hbm.at[pl.ds(0, GRPR), :], wv[b],
                              semW[b]).wait()
        pltpu.make_async_copy(u_hbm.at[pl.ds(0, GRP), :], rowsU[b],
                              semG[b]).wait()

    def compute_scatter(b):
        ru, w_ = rowsU[b], wv[b]

        @plsc.parallel_loop(0, GRPR, step=1, unroll=2)
        def _(i):
            for j in range(8):
                ru[8 * i + j] = ru[8 * i + j] + w_[i, pl.ds(j * F, F)]

        for j in range(KJ):
            pltpu.async_copy(ru.at[pl.ds(j * CH, CH), :],
                             acc.at[idxd[b].at[j]], semS[b], add=True)

    def wait_scat(b):
        pltpu.make_async_copy(u_hbm.at[pl.ds(0, GRP), :], rowsU[b],
                              semS[b]).wait()

    fire_idx(0, 0)
    wait_idx(0)
    fire_grp(0, 0)
    fire_idx(1, 1)

    def pair(k, _):
        g0 = 2 * k
        wait_idx(1)
        fire_grp(g0 + 1, 1)
        wait_grp(0)
        compute_scatter(0)
        wait_grp(1)
        compute_scatter(1)

        @pl.when(k < G // 2 - 1)
        def _():
            wait_scat(0)
            fire_idx(g0 + 2, 0)
            wait_idx(0)
            fire_grp(g0 + 2, 0)
            wait_scat(1)
            fire_idx(g0 + 3, 1)

        return 0

    lax.fori_loop(0, G // 2, pair, 0)
    wait_scat(0)
    wait_scat(1)
    plsc.subcore_barrier()

    for r in range(PER_SUB // ZB):
        pltpu.sync_copy(acc.at[pl.ds(sid * PER_SUB + r * ZB, ZB), :], zbuf)
        pltpu.sync_copy(zbuf,
                        out_hbm.at[cid, pl.ds(sid * PER_SUB + r * ZB, ZB), :])


def kernel(nte, ete, neW1, neb1, neW2, neb2, eeW1, eeb1, eeW2, eeb2,
           ceW1, ceb1, ceW2, ceb2, cnW1, cnb1, cnW2, cnb2,
           ndW1, ndb1, ndW2, ndb2, ndW3, ndb3,
           edW1, edb1, edW2, edb2, edW3, edb3,
           x_node_types, x_edge_types, edge_index, steps):
    relu = _relu

    def mlp2(x, W1, b1, W2, b2):
        return relu(relu(x @ W1 + b1) @ W2 + b2)

    ntab = mlp2(nte, neW1, neb1, neW2, neb2)
    etab = mlp2(ete, eeW1, eeb1, eeW2, eeb2)
    Wa1, Wa2 = ceW1[0:16], ceW1[16:32]
    Wb1, Wb2 = ceW1[32:48], ceW1[48:64]
    Wc1, Wc2 = ceW1[64:80], ceW1[80:96]
    Wu1, Wu2 = cnW1[0:16], cnW1[16:32]
    M2 = cnW1[32:48]

    def pad8(t):
        return jnp.pad(t, ((0, 8 - t.shape[0]), (0, 0)))

    TN = pad8(jnp.concatenate([ntab @ Wa1, ntab @ Wc1, ntab @ Wu1], axis=1))
    T0 = pad8(jnp.concatenate([ntab @ (Wa1 + Wa2), ntab @ (Wc1 + Wc2),
                               ntab @ (Wu1 + Wu2)], axis=1))
    W3 = jnp.concatenate([Wa2, Wc2, Wu2], axis=1)

    eye8 = jnp.eye(8, dtype=_f32)
    kr = lambda W: jnp.kron(eye8, W)
    t8 = lambda b: jnp.tile(b.reshape(1, -1), (1, 8))
    Tbt = pad8(jnp.tile(etab @ Wb1 + ceb1, (1, 8)))
    Tb0t = pad8(jnp.tile(etab @ (Wb1 + Wb2) + ceb1, (1, 8)))
    ceW2d, M2d, Wb2d = kr(ceW2), kr(M2), kr(Wb2)
    ceb2t = t8(ceb2)
    edW1d, edW2d = kr(edW1), kr(edW2)
    edW3d = kr(jnp.pad(edW3, ((0, 0), (0, 8 - edW3.shape[1]))))
    edb1t, edb2t = t8(edb1), t8(edb2)
    edb3t = t8(jnp.pad(edb3, (0, 8 - edb3.shape[0])))

    row = lambda b: b.reshape(1, F)
    cnb1r, cnb2r = row(cnb1), row(cnb2)
    ndb1r, ndb2r = row(ndb1), row(ndb2)
    ndW3p = jnp.pad(ndW3, ((0, 0), (0, 8 - ndW3.shape[1])))
    ndb3p = jnp.pad(ndb3, (0, 8 - ndb3.shape[0])).reshape(1, 8)

    src = edge_index[0].astype(_i32)
    dst = edge_index[1].astype(_i32)
    padE = E_PAD - E
    src2 = jnp.concatenate([src, jnp.full((padE,), DUMMY, _i32)]
                           ).reshape(E_PAD // CH, CH)
    dst2 = jnp.concatenate(
        [dst, DUMMY + (jnp.arange(padE, dtype=_i32) % (N_TAB - N))]
    ).reshape(E_PAD // CH, CH)
    ntf = jnp.pad(x_node_types.astype(_f32), (0, N_TAB - N)
                  ).reshape(N_TAB, 1)
    etr = jnp.repeat(jnp.pad(x_edge_types.astype(_f32), (0, padE)), F
                     ).reshape(EP8, 128)

    a0, c0, u0 = _tc_init_nodes(ntf, T0)
    eb0 = _tc_init_edges(etr, Tb0t)
    er0 = jnp.zeros((EP8, 128), _f32)
    xn0 = jnp.zeros((N_TAB, F), _f32)

    def body(t, carry):
        a, c, u, eb, er, xn = carry
        er1 = _sc_gather_kernel()(a, c, eb, src2, dst2)
        er, w, ebn = _tc_edge(er1, etr, ceW2d, ceb2t, M2d, Wb2d, Tbt)
        S2 = _sc_scatter_kernel()(u, w, src2, dst2)
        xn, a, c, u = _tc_node(S2[0], S2[1], ntf, cnb1r, cnW2, cnb2r, TN, W3)
        return (a, c, u, ebn, er, xn)

    a, c, u, eb, er, xn = lax.fori_loop(
        0, steps, body, (a0, c0, u0, eb0, er0, xn0))

    pn = _tc_dec(xn, ndW1, ndb1r, ndW2, ndb2r, ndW3p, ndb3p, N_TAB, BN)
    pep = _tc_dec_edge(er, edW1d, edb1t, edW2d, edb2t, edW3d, edb3t)
    pe = pep.reshape(E_PAD, 8)
    return (pn[:N, :3], pe[:E, :3])


# speedup vs baseline: 1.9003x; 1.0110x over previous
import functools

import jax
import jax.numpy as jnp
from jax import lax
from jax.experimental import pallas as pl
from jax.experimental.pallas import tpu as pltpu
from jax.experimental.pallas import tpu_sc as plsc

F = 16
NCORE = 2
NSUB = 16
NW = NCORE * NSUB
CH = 128
KJ = 8
GRP = KJ * CH

N = 50000
E = 800000
G = -(-(E // NW) // GRP)
G = G + (G % 2)
E_PAD = NW * G * GRP
EP8 = E_PAD // 8
GRPR = GRP // 8
N_TAB = 50176
PER_SUB = N_TAB // NSUB
ZB = 392
DUMMY = N

BN = 1024
BRE = 512

_f32 = jnp.float32
_i32 = jnp.int32


def _relu(x):
    return jnp.maximum(x, 0.0)


def _onehot(tf_ref, rows):
    return (tf_ref[...].astype(_i32)
            == lax.broadcasted_iota(_i32, (rows, 8), 1)).astype(_f32)


def _tc_init_nodes(ntf, T0):
    def body(ntf_ref, T0_ref, a_ref, c_ref, u_ref):
        acu = jnp.dot(_onehot(ntf_ref, BN), T0_ref[...],
                      preferred_element_type=_f32)
        a_ref[...] = acu[:, 0:16]
        c_ref[...] = acu[:, 16:32]
        u_ref[...] = acu[:, 32:48]

    o = jax.ShapeDtypeStruct((N_TAB, F), _f32)
    return pl.pallas_call(
        body,
        grid=(N_TAB // BN,),
        in_specs=[pl.BlockSpec((BN, 1), lambda i: (i, 0)),
                  pl.BlockSpec((8, 48), lambda i: (0, 0))],
        out_specs=[pl.BlockSpec((BN, F), lambda i: (i, 0))] * 3,
        out_shape=[o, o, o],
    )(ntf, T0)


def _type_mix(et, T_ref):
    return (jnp.where(et == 0.0, T_ref[0:1, :], 0.0)
            + jnp.where(et == 1.0, T_ref[1:2, :], 0.0)
            + jnp.where(et == 2.0, T_ref[2:3, :], 0.0))


def _tc_init_edges(etr, Tb0t):
    def body(et_ref, Tb0_ref, eb_ref):
        eb_ref[...] = _type_mix(et_ref[...], Tb0_ref)

    return pl.pallas_call(
        body,
        grid=(EP8 // BRE,),
        in_specs=[pl.BlockSpec((BRE, 128), lambda i: (i, 0)),
                  pl.BlockSpec((8, 128), lambda i: (0, 0))],
        out_specs=pl.BlockSpec((BRE, 128), lambda i: (i, 0)),
        out_shape=jax.ShapeDtypeStruct((EP8, 128), _f32),
    )(etr, Tb0t)


def _tc_edge(er1p, etr, W2d, b2t, M2d, Wb2d, Tbt):
    def body(er1_ref, et_ref, W2_ref, b2_ref, M2_ref, Wb2_ref, Tb_ref,
             w_ref, ebn_ref):
        er = _relu(jnp.dot(er1_ref[...], W2_ref[...],
                           preferred_element_type=_f32) + b2_ref[...])
        w_ref[...] = jnp.dot(er, M2_ref[...], preferred_element_type=_f32)
        ebn_ref[...] = (_type_mix(et_ref[...], Tb_ref)
                        + jnp.dot(er, Wb2_ref[...],
                                  preferred_element_type=_f32))

    full = lambda shape: pl.BlockSpec(shape, lambda i: (0, 0))
    o = jax.ShapeDtypeStruct((EP8, 128), _f32)
    return pl.pallas_call(
        body,
        grid=(EP8 // BRE,),
        in_specs=[pl.BlockSpec((BRE, 128), lambda i: (i, 0)),
                  pl.BlockSpec((BRE, 128), lambda i: (i, 0)),
                  full((128, 128)), full((1, 128)), full((128, 128)),
                  full((128, 128)), full((8, 128))],
        out_specs=[pl.BlockSpec((BRE, 128), lambda i: (i, 0))] * 2,
        out_shape=[o, o],
    )(er1p, etr, W2d, b2t, M2d, Wb2d, Tbt)


def _tc_node(S0, S1, ntf, cnb1, cnW2, cnb2, TN, W3):
    def body(s0_ref, s1_ref, ntf_ref, b1_ref, W2_ref, b2_ref, TN_ref, W3_ref,
             xn_ref, a_ref, c_ref, u_ref):
        xn1 = _relu(s0_ref[...] + s1_ref[...] + b1_ref[...])
        xn = _relu(jnp.dot(xn1, W2_ref[...], preferred_element_type=_f32)
                   + b2_ref[...])
        xn_ref[...] = xn
        acu = (jnp.dot(_onehot(ntf_ref, BN), TN_ref[...],
                       preferred_element_type=_f32)
               + jnp.dot(xn, W3_ref[...], preferred_element_type=_f32))
        a_ref[...] = acu[:, 0:16]
        c_ref[...] = acu[:, 16:32]
        u_ref[...] = acu[:, 32:48]

    full = lambda shape: pl.BlockSpec(shape, lambda i: (0, 0))
    o = jax.ShapeDtypeStruct((N_TAB, F), _f32)
    return pl.pallas_call(
        body,
        grid=(N_TAB // BN,),
        in_specs=[pl.BlockSpec((BN, F), lambda i: (i, 0)),
                  pl.BlockSpec((BN, F), lambda i: (i, 0)),
                  pl.BlockSpec((BN, 1), lambda i: (i, 0)),
                  full((1, F)), full((F, F)), full((1, F)),
                  full((8, 48)), full((F, 48))],
        out_specs=[pl.BlockSpec((BN, F), lambda i: (i, 0))] * 4,
        out_shape=[o, o, o, o],
    )(S0, S1, ntf, cnb1, cnW2, cnb2, TN, W3)


def _tc_dec(x, W1, b1, W2, b2, W3p, b3p, rows, block):
    def body(x_ref, W1_ref, b1_ref, W2_ref, b2_ref, W3_ref, b3_ref, o_ref):
        h = _relu(jnp.dot(x_ref[...], W1_ref[...],
                          preferred_element_type=_f32) + b1_ref[...])
        h = _relu(jnp.dot(h, W2_ref[...], preferred_element_type=_f32)
                  + b2_ref[...])
        o_ref[...] = jnp.dot(h, W3_ref[...],
                             preferred_element_type=_f32) + b3_ref[...]

    full = lambda shape: pl.BlockSpec(shape, lambda i: (0, 0))
    return pl.pallas_call(
        body,
        grid=(rows // block,),
        in_specs=[pl.BlockSpec((block, F), lambda i: (i, 0)),
                  full((F, F)), full((1, F)), full((F, F)), full((1, F)),
                  full((F, 8)), full((1, 8))],
        out_specs=pl.BlockSpec((block, 8), lambda i: (i, 0)),
        out_shape=jax.ShapeDtypeStruct((rows, 8), _f32),
    )(x, W1, b1, W2, b2, W3p, b3p)


def _tc_dec_edge(er1p, eW2d, eb2t, W1d, b1t, W2d, b2t, W3d, b3t):
    def body(x_ref, eW2_ref, eb2_ref, W1_ref, b1_ref, W2_ref, b2_ref,
             W3_ref, b3_ref, o_ref):
        er = _relu(jnp.dot(x_ref[...], eW2_ref[...],
                           preferred_element_type=_f32) + eb2_ref[...])
        h = _relu(jnp.dot(er, W1_ref[...],
                          preferred_element_type=_f32) + b1_ref[...])
        h = _relu(jnp.dot(h, W2_ref[...], preferred_element_type=_f32)
                  + b2_ref[...])
        o_ref[...] = jnp.dot(h, W3_ref[...],
                             preferred_element_type=_f32) + b3_ref[...]

    full = lambda shape: pl.BlockSpec(shape, lambda i: (0, 0))
    return pl.pallas_call(
        body,
        grid=(EP8 // BRE,),
        in_specs=[pl.BlockSpec((BRE, 128), lambda i: (i, 0)),
                  full((128, 128)), full((1, 128)),
                  full((128, 128)), full((1, 128)), full((128, 128)),
                  full((1, 128)), full((128, 64)), full((1, 64))],
        out_specs=pl.BlockSpec((BRE, 64), lambda i: (i, 0)),
        out_shape=jax.ShapeDtypeStruct((EP8, 64), _f32),
    )(er1p, eW2d, eb2t, W1d, b1t, W2d, b2t, W3d, b3t)


@functools.lru_cache(maxsize=None)
def _sc_gather_kernel():
    mesh = plsc.VectorSubcoreMesh(core_axis_name="c", subcore_axis_name="s")
    buf = [pltpu.VMEM((GRP,), _i32),
           pltpu.VMEM((GRP,), _i32),
           pltpu.VMEM((GRP, F), _f32),
           pltpu.VMEM((GRP, F), _f32),
           pltpu.VMEM((GRPR, 128), _f32),
           pltpu.SemaphoreType.DMA,
           pltpu.SemaphoreType.DMA,
           pltpu.SemaphoreType.DMA,
           pltpu.SemaphoreType.DMA]
    return functools.partial(
        pl.kernel, mesh=mesh,
        compiler_params=pltpu.CompilerParams(use_tc_tiling_on_sc=False),
        out_type=jax.ShapeDtypeStruct((EP8, 128), _f32),
        scratch_types=buf + buf)(_sc_gather_body)


def _sc_gather_body(a_hbm, c_hbm, eb_hbm, src1_hbm, dst1_hbm, out_hbm, *s):
    idxs = (s[0], s[9])
    idxd = (s[1], s[10])
    rowsA = (s[2], s[11])
    rowsC = (s[3], s[12])
    ebv = (s[4], s[13])
    semI = (s[5], s[14])
    semG = (s[6], s[15])
    semE = (s[7], s[16])
    semO = (s[8], s[17])
    wid = lax.axis_index("s") * NCORE + lax.axis_index("c")
    base_g = wid * G

    def fire_idx(g, b):
        off = (base_g + g) * GRP
        pltpu.async_copy(src1_hbm.at[pl.ds(off, GRP)], idxs[b], semI[b])
        pltpu.async_copy(dst1_hbm.at[pl.ds(off, GRP)], idxd[b], semI[b])

    def wait_idx(b):
        pltpu.make_async_copy(src1_hbm.at[pl.ds(0, GRP)], idxs[b],
                              semI[b]).wait()
        pltpu.make_async_copy(dst1_hbm.at[pl.ds(0, GRP)], idxd[b],
                              semI[b]).wait()

    def fire_grp(g, b):
        offp = (base_g + g) * GRPR
        pltpu.async_copy(eb_hbm.at[pl.ds(offp, GRPR), :], ebv[b], semE[b])
        pltpu.async_copy(a_hbm.at[idxs[b]], rowsA[b], semG[b])
        pltpu.async_copy(c_hbm.at[idxd[b]], rowsC[b], semG[b])

    def wait_grp(b):
        pltpu.make_async_copy(eb_hbm.at[pl.ds(0, GRPR), :], ebv[b],
                              semE[b]).wait()
        pltpu.make_async_copy(a_hbm.at[pl.ds(0, GRP), :], rowsA[b],
                              semG[b]).wait()
        pltpu.make_async_copy(a_hbm.at[pl.ds(0, GRP), :], rowsC[b],
                              semG[b]).wait()

    def compute_store(g, b):
        ra, rc, eb_ = rowsA[b], rowsC[b], ebv[b]

        @plsc.parallel_loop(0, GRPR, step=1, unroll=2)
        def _(i):
            for j in range(8):
                sl = pl.ds(j * F, F)
                eb_[i, sl] = jnp.maximum(
                    ra[8 * i + j] + rc[8 * i + j] + eb_[i, sl], 0.0)

        offp = (base_g + g) * GRPR
        pltpu.async_copy(eb_, out_hbm.at[pl.ds(offp, GRPR), :], semO[b])

    def wait_store(b):
        pltpu.make_async_copy(ebv[b], out_hbm.at[pl.ds(0, GRPR), :],
                              semO[b]).wait()

    fire_idx(0, 0)
    wait_idx(0)
    fire_grp(0, 0)
    fire_idx(1, 1)

    def pair(k, _):
        g0 = 2 * k
        wait_idx(1)
        fire_grp(g0 + 1, 1)
        wait_grp(0)
        compute_store(g0, 0)
        wait_grp(1)
        compute_store(g0 + 1, 1)

        @pl.when(k < G // 2 - 1)
        def _():
            wait_store(0)
            fire_idx(g0 + 2, 0)
            wait_idx(0)
            fire_grp(g0 + 2, 0)
            wait_store(1)
            fire_idx(g0 + 3, 1)

        return 0

    lax.fori_loop(0, G // 2, pair, 0)
    wait_store(0)
    wait_store(1)


@functools.lru_cache(maxsize=None)
def _sc_scatter_kernel():
    mesh = plsc.VectorSubcoreMesh(core_axis_name="c", subcore_axis_name="s")
    buf = [pltpu.VMEM((GRP,), _i32),
           pltpu.VMEM((GRP,), _i32),
           pltpu.VMEM((GRP, F), _f32),
           pltpu.VMEM((GRPR, 128), _f32),
           pltpu.SemaphoreType.DMA,
           pltpu.SemaphoreType.DMA,
           pltpu.SemaphoreType.DMA,
           pltpu.SemaphoreType.DMA]
    return functools.partial(
        pl.kernel, mesh=mesh,
        compiler_params=pltpu.CompilerParams(use_tc_tiling_on_sc=False),
        out_type=jax.ShapeDtypeStruct((NCORE, N_TAB, F), _f32),
        scratch_types=buf + buf + [
            pltpu.VMEM((ZB, F), _f32),
            pltpu.VMEM_SHARED((N_TAB, F), _f32),
        ])(_sc_scatter_body)


def _sc_scatter_body(u_hbm, w_hbm, src1_hbm, dst1_hbm, out_hbm, *s):
    idxs = (s[0], s[8])
    idxd = (s[1], s[9])
    rowsU = (s[2], s[10])
    wv = (s[3], s[11])
    semI = (s[4], s[12])
    semG = (s[5], s[13])
    semW = (s[6], s[14])
    semS = (s[7], s[15])
    zbuf = s[16]
    acc = s[17]
    cid = lax.axis_index("c")
    sid = lax.axis_index("s")
    wid = sid * NCORE + cid
    base_g = wid * G

    @plsc.parallel_loop(0, ZB, step=1, unroll=8)
    def _(i):
        zbuf[i] = jnp.zeros((F,), _f32)

    for r in range(PER_SUB // ZB):
        pltpu.sync_copy(zbuf, acc.at[pl.ds(sid * PER_SUB + r * ZB, ZB), :])
    plsc.subcore_barrier()

    def fire_idx(g, b):
        off = (base_g + g) * GRP
        pltpu.async_copy(src1_hbm.at[pl.ds(off, GRP)], idxs[b], semI[b])
        pltpu.async_copy(dst1_hbm.at[pl.ds(off, GRP)], idxd[b], semI[b])

    def wait_idx(b):
        pltpu.make_async_copy(src1_hbm.at[pl.ds(0, GRP)], idxs[b],
                              semI[b]).wait()
        pltpu.make_async_copy(dst1_hbm.at[pl.ds(0, GRP)], idxd[b],
                              semI[b]).wait()

    def fire_grp(g, b):
        offp = (base_g + g) * GRPR
        pltpu.async_copy(w_hbm.at[pl.ds(offp, GRPR), :], wv[b], semW[b])
        pltpu.async_copy(u_hbm.at[idxs[b]], rowsU[b], semG[b])

    def wait_grp(b):
        pltpu.make_async_copy(w_hbm.at[pl.ds(0, GRPR), :], wv[b],
                              semW[b]).wait()
        pltpu.make_async_copy(u_hbm.at[pl.ds(0, GRP), :], rowsU[b],
                              semG[b]).wait()

    def compute_scatter(b):
        ru, w_ = rowsU[b], wv[b]

        @plsc.parallel_loop(0, GRPR, step=1, unroll=2)
        def _(i):
            for j in range(8):
                ru[8 * i + j] = ru[8 * i + j] + w_[i, pl.ds(j * F, F)]

        pltpu.async_copy(ru, acc.at[idxd[b]], semS[b], add=True)

    def wait_scat(b):
        pltpu.make_async_copy(u_hbm.at[pl.ds(0, GRP), :], rowsU[b],
                              semS[b]).wait()

    fire_idx(0, 0)
    wait_idx(0)
    fire_grp(0, 0)
    fire_idx(1, 1)

    def pair(k, _):
        g0 = 2 * k
        wait_idx(1)
        fire_grp(g0 + 1, 1)
        wait_grp(0)
        compute_scatter(0)
        wait_grp(1)
        compute_scatter(1)

        @pl.when(k < G // 2 - 1)
        def _():
            wait_scat(0)
            fire_idx(g0 + 2, 0)
            wait_idx(0)
            fire_grp(g0 + 2, 0)
            wait_scat(1)
            fire_idx(g0 + 3, 1)

        return 0

    lax.fori_loop(0, G // 2, pair, 0)
    wait_scat(0)
    wait_scat(1)
    plsc.subcore_barrier()

    for r in range(PER_SUB // ZB):
        pltpu.sync_copy(acc.at[pl.ds(sid * PER_SUB + r * ZB, ZB), :], zbuf)
        pltpu.sync_copy(zbuf,
                        out_hbm.at[cid, pl.ds(sid * PER_SUB + r * ZB, ZB), :])


def kernel(nte, ete, neW1, neb1, neW2, neb2, eeW1, eeb1, eeW2, eeb2,
           ceW1, ceb1, ceW2, ceb2, cnW1, cnb1, cnW2, cnb2,
           ndW1, ndb1, ndW2, ndb2, ndW3, ndb3,
           edW1, edb1, edW2, edb2, edW3, edb3,
           x_node_types, x_edge_types, edge_index, steps):
    relu = _relu

    def mlp2(x, W1, b1, W2, b2):
        return relu(relu(x @ W1 + b1) @ W2 + b2)

    ntab = mlp2(nte, neW1, neb1, neW2, neb2)
    etab = mlp2(ete, eeW1, eeb1, eeW2, eeb2)
    Wa1, Wa2 = ceW1[0:16], ceW1[16:32]
    Wb1, Wb2 = ceW1[32:48], ceW1[48:64]
    Wc1, Wc2 = ceW1[64:80], ceW1[80:96]
    Wu1, Wu2 = cnW1[0:16], cnW1[16:32]
    M2 = cnW1[32:48]

    def pad8(t):
        return jnp.pad(t, ((0, 8 - t.shape[0]), (0, 0)))

    TN = pad8(jnp.concatenate([ntab @ Wa1, ntab @ Wc1, ntab @ Wu1], axis=1))
    T0 = pad8(jnp.concatenate([ntab @ (Wa1 + Wa2), ntab @ (Wc1 + Wc2),
                               ntab @ (Wu1 + Wu2)], axis=1))
    W3 = jnp.concatenate([Wa2, Wc2, Wu2], axis=1)

    eye8 = jnp.eye(8, dtype=_f32)
    kr = lambda W: jnp.kron(eye8, W)
    t8 = lambda b: jnp.tile(b.reshape(1, -1), (1, 8))
    Tbt = pad8(jnp.tile(etab @ Wb1 + ceb1, (1, 8)))
    Tb0t = pad8(jnp.tile(etab @ (Wb1 + Wb2) + ceb1, (1, 8)))
    ceW2d, M2d, Wb2d = kr(ceW2), kr(M2), kr(Wb2)
    ceb2t = t8(ceb2)
    edW1d, edW2d = kr(edW1), kr(edW2)
    edW3d = kr(jnp.pad(edW3, ((0, 0), (0, 8 - edW3.shape[1]))))
    edb1t, edb2t = t8(edb1), t8(edb2)
    edb3t = t8(jnp.pad(edb3, (0, 8 - edb3.shape[0])))

    row = lambda b: b.reshape(1, F)
    cnb1r, cnb2r = row(cnb1), row(cnb2)
    ndb1r, ndb2r = row(ndb1), row(ndb2)
    ndW3p = jnp.pad(ndW3, ((0, 0), (0, 8 - ndW3.shape[1])))
    ndb3p = jnp.pad(ndb3, (0, 8 - ndb3.shape[0])).reshape(1, 8)

    src = edge_index[0].astype(_i32)
    dst = edge_index[1].astype(_i32)
    padE = E_PAD - E
    src1 = jnp.concatenate([src, jnp.full((padE,), DUMMY, _i32)])
    dst1 = jnp.concatenate(
        [dst, DUMMY + (jnp.arange(padE, dtype=_i32) % (N_TAB - N))])
    ntf = jnp.pad(x_node_types.astype(_f32), (0, N_TAB - N)
                  ).reshape(N_TAB, 1)
    etr = jnp.repeat(jnp.pad(x_edge_types.astype(_f32), (0, padE)), F
                     ).reshape(EP8, 128)

    a0, c0, u0 = _tc_init_nodes(ntf, T0)
    eb0 = _tc_init_edges(etr, Tb0t)
    er0 = jnp.zeros((EP8, 128), _f32)
    xn0 = jnp.zeros((N_TAB, F), _f32)

    def body(t, carry):
        a, c, u, eb, er1, xn = carry
        er1 = _sc_gather_kernel()(a, c, eb, src1, dst1)
        w, ebn = _tc_edge(er1, etr, ceW2d, ceb2t, M2d, Wb2d, Tbt)
        S2 = _sc_scatter_kernel()(u, w, src1, dst1)
        xn, a, c, u = _tc_node(S2[0], S2[1], ntf, cnb1r, cnW2, cnb2r, TN, W3)
        return (a, c, u, ebn, er1, xn)

    a, c, u, eb, er1, xn = lax.fori_loop(
        0, steps, body, (a0, c0, u0, eb0, er0, xn0))

    pn = _tc_dec(xn, ndW1, ndb1r, ndW2, ndb2r, ndW3p, ndb3p, N_TAB, BN)
    pep = _tc_dec_edge(er1, ceW2d, ceb2t, edW1d, edb1t, edW2d, edb2t,
                       edW3d, edb3t)
    pe = pep.reshape(E_PAD, 8)
    return (pn[:N, :3], pe[:E, :3])


# speedup vs baseline: 1.9018x; 1.0008x over previous
import functools

import jax
import jax.numpy as jnp
from jax import lax
from jax.experimental import pallas as pl
from jax.experimental.pallas import tpu as pltpu
from jax.experimental.pallas import tpu_sc as plsc

F = 16
NCORE = 2
NSUB = 16
NW = NCORE * NSUB
CH = 128
KJ = 8
GRP = KJ * CH

N = 50000
E = 800000
G = -(-(E // NW) // GRP)
G = G + (G % 2)
E_PAD = NW * G * GRP
EP8 = E_PAD // 8
GRPR = GRP // 8
N_TAB = 50176
PER_SUB = N_TAB // NSUB
ZB = 392
DUMMY = N

BN = 1024
BRE = 512

_f32 = jnp.float32
_i32 = jnp.int32


def _relu(x):
    return jnp.maximum(x, 0.0)


def _onehot(tf_ref, rows):
    return (tf_ref[...].astype(_i32)
            == lax.broadcasted_iota(_i32, (rows, 8), 1)).astype(_f32)


def _tc_init_nodes(ntf, T0):
    def body(ntf_ref, T0_ref, a_ref, c_ref, u_ref):
        acu = jnp.dot(_onehot(ntf_ref, BN), T0_ref[...],
                      preferred_element_type=_f32)
        a_ref[...] = acu[:, 0:16]
        c_ref[...] = acu[:, 16:32]
        u_ref[...] = acu[:, 32:48]

    o = jax.ShapeDtypeStruct((N_TAB, F), _f32)
    return pl.pallas_call(
        body,
        grid=(N_TAB // BN,),
        in_specs=[pl.BlockSpec((BN, 1), lambda i: (i, 0)),
                  pl.BlockSpec((8, 48), lambda i: (0, 0))],
        out_specs=[pl.BlockSpec((BN, F), lambda i: (i, 0))] * 3,
        out_shape=[o, o, o],
    )(ntf, T0)


def _type_mix(et, T_ref):
    return (jnp.where(et == 0.0, T_ref[0:1, :], 0.0)
            + jnp.where(et == 1.0, T_ref[1:2, :], 0.0)
            + jnp.where(et == 2.0, T_ref[2:3, :], 0.0))


def _tc_init_edges(etr, Tb0t):
    def body(et_ref, Tb0_ref, eb_ref):
        eb_ref[...] = _type_mix(et_ref[...], Tb0_ref)

    return pl.pallas_call(
        body,
        grid=(EP8 // BRE,),
        in_specs=[pl.BlockSpec((BRE, 128), lambda i: (i, 0)),
                  pl.BlockSpec((8, 128), lambda i: (0, 0))],
        out_specs=pl.BlockSpec((BRE, 128), lambda i: (i, 0)),
        out_shape=jax.ShapeDtypeStruct((EP8, 128), _f32),
    )(etr, Tb0t)


def _tc_edge(er1p, etr, W2d, b2t, M2d, Wb2d, Tbt):
    def body(er1_ref, et_ref, W2_ref, b2_ref, M2_ref, Wb2_ref, Tb_ref,
             w_ref, ebn_ref):
        er = _relu(jnp.dot(er1_ref[...], W2_ref[...],
                           preferred_element_type=_f32) + b2_ref[...])
        w_ref[...] = jnp.dot(er, M2_ref[...], preferred_element_type=_f32)
        ebn_ref[...] = (_type_mix(et_ref[...], Tb_ref)
                        + jnp.dot(er, Wb2_ref[...],
                                  preferred_element_type=_f32))

    full = lambda shape: pl.BlockSpec(shape, lambda i: (0, 0))
    o = jax.ShapeDtypeStruct((EP8, 128), _f32)
    return pl.pallas_call(
        body,
        grid=(EP8 // BRE,),
        in_specs=[pl.BlockSpec((BRE, 128), lambda i: (i, 0)),
                  pl.BlockSpec((BRE, 128), lambda i: (i, 0)),
                  full((128, 128)), full((1, 128)), full((128, 128)),
                  full((128, 128)), full((8, 128))],
        out_specs=[pl.BlockSpec((BRE, 128), lambda i: (i, 0))] * 2,
        out_shape=[o, o],
    )(er1p, etr, W2d, b2t, M2d, Wb2d, Tbt)


def _tc_node(S0, S1, ntf, cnb1, cnW2, cnb2, TN, W3):
    def body(s0_ref, s1_ref, ntf_ref, b1_ref, W2_ref, b2_ref, TN_ref, W3_ref,
             xn_ref, a_ref, c_ref, u_ref):
        xn1 = _relu(s0_ref[...] + s1_ref[...] + b1_ref[...])
        xn = _relu(jnp.dot(xn1, W2_ref[...], preferred_element_type=_f32)
                   + b2_ref[...])
        xn_ref[...] = xn
        acu = (jnp.dot(_onehot(ntf_ref, BN), TN_ref[...],
                       preferred_element_type=_f32)
               + jnp.dot(xn, W3_ref[...], preferred_element_type=_f32))
        a_ref[...] = acu[:, 0:16]
        c_ref[...] = acu[:, 16:32]
        u_ref[...] = acu[:, 32:48]

    full = lambda shape: pl.BlockSpec(shape, lambda i: (0, 0))
    o = jax.ShapeDtypeStruct((N_TAB, F), _f32)
    return pl.pallas_call(
        body,
        grid=(N_TAB // BN,),
        in_specs=[pl.BlockSpec((BN, F), lambda i: (i, 0)),
                  pl.BlockSpec((BN, F), lambda i: (i, 0)),
                  pl.BlockSpec((BN, 1), lambda i: (i, 0)),
                  full((1, F)), full((F, F)), full((1, F)),
                  full((8, 48)), full((F, 48))],
        out_specs=[pl.BlockSpec((BN, F), lambda i: (i, 0))] * 4,
        out_shape=[o, o, o, o],
    )(S0, S1, ntf, cnb1, cnW2, cnb2, TN, W3)


def _tc_dec(x, W1, b1, W2, b2, W3p, b3p, rows, block):
    def body(x_ref, W1_ref, b1_ref, W2_ref, b2_ref, W3_ref, b3_ref, o_ref):
        h = _relu(jnp.dot(x_ref[...], W1_ref[...],
                          preferred_element_type=_f32) + b1_ref[...])
        h = _relu(jnp.dot(h, W2_ref[...], preferred_element_type=_f32)
                  + b2_ref[...])
        o_ref[...] = jnp.dot(h, W3_ref[...],
                             preferred_element_type=_f32) + b3_ref[...]

    full = lambda shape: pl.BlockSpec(shape, lambda i: (0, 0))
    return pl.pallas_call(
        body,
        grid=(rows // block,),
        in_specs=[pl.BlockSpec((block, F), lambda i: (i, 0)),
                  full((F, F)), full((1, F)), full((F, F)), full((1, F)),
                  full((F, 8)), full((1, 8))],
        out_specs=pl.BlockSpec((block, 8), lambda i: (i, 0)),
        out_shape=jax.ShapeDtypeStruct((rows, 8), _f32),
    )(x, W1, b1, W2, b2, W3p, b3p)


def _tc_dec_edge(er1p, eW2d, eb2t, W1d, b1t, W2d, b2t, W3d, b3t):
    def body(x_ref, eW2_ref, eb2_ref, W1_ref, b1_ref, W2_ref, b2_ref,
             W3_ref, b3_ref, o_ref):
        er = _relu(jnp.dot(x_ref[...], eW2_ref[...],
                           preferred_element_type=_f32) + eb2_ref[...])
        h = _relu(jnp.dot(er, W1_ref[...],
                          preferred_element_type=_f32) + b1_ref[...])
        h = _relu(jnp.dot(h, W2_ref[...], preferred_element_type=_f32)
                  + b2_ref[...])
        o_ref[...] = jnp.dot(h, W3_ref[...],
                             preferred_element_type=_f32) + b3_ref[...]

    full = lambda shape: pl.BlockSpec(shape, lambda i: (0, 0))
    return pl.pallas_call(
        body,
        grid=(EP8 // BRE,),
        in_specs=[pl.BlockSpec((BRE, 128), lambda i: (i, 0)),
                  full((128, 128)), full((1, 128)),
                  full((128, 128)), full((1, 128)), full((128, 128)),
                  full((1, 128)), full((128, 128)), full((1, 128))],
        out_specs=pl.BlockSpec((BRE, 128), lambda i: (i, 0)),
        out_shape=jax.ShapeDtypeStruct((EP8, 128), _f32),
    )(er1p, eW2d, eb2t, W1d, b1t, W2d, b2t, W3d, b3t)


@functools.lru_cache(maxsize=None)
def _sc_gather_kernel():
    mesh = plsc.VectorSubcoreMesh(core_axis_name="c", subcore_axis_name="s")
    buf = [pltpu.VMEM((GRP,), _i32),
           pltpu.VMEM((GRP,), _i32),
           pltpu.VMEM((GRP, F), _f32),
           pltpu.VMEM((GRP, F), _f32),
           pltpu.VMEM((GRPR, 128), _f32),
           pltpu.SemaphoreType.DMA,
           pltpu.SemaphoreType.DMA,
           pltpu.SemaphoreType.DMA,
           pltpu.SemaphoreType.DMA]
    return functools.partial(
        pl.kernel, mesh=mesh,
        compiler_params=pltpu.CompilerParams(use_tc_tiling_on_sc=False),
        out_type=jax.ShapeDtypeStruct((EP8, 128), _f32),
        scratch_types=buf + buf)(_sc_gather_body)


def _sc_gather_body(a_hbm, c_hbm, eb_hbm, src1_hbm, dst1_hbm, out_hbm, *s):
    idxs = (s[0], s[9])
    idxd = (s[1], s[10])
    rowsA = (s[2], s[11])
    rowsC = (s[3], s[12])
    ebv = (s[4], s[13])
    semI = (s[5], s[14])
    semG = (s[6], s[15])
    semE = (s[7], s[16])
    semO = (s[8], s[17])
    wid = lax.axis_index("s") * NCORE + lax.axis_index("c")
    base_g = wid * G

    def fire_idx(g, b):
        off = (base_g + g) * GRP
        pltpu.async_copy(src1_hbm.at[pl.ds(off, GRP)], idxs[b], semI[b])
        pltpu.async_copy(dst1_hbm.at[pl.ds(off, GRP)], idxd[b], semI[b])

    def wait_idx(b):
        pltpu.make_async_copy(src1_hbm.at[pl.ds(0, GRP)], idxs[b],
                              semI[b]).wait()
        pltpu.make_async_copy(dst1_hbm.at[pl.ds(0, GRP)], idxd[b],
                              semI[b]).wait()

    def fire_grp(g, b):
        offp = (base_g + g) * GRPR
        pltpu.async_copy(eb_hbm.at[pl.ds(offp, GRPR), :], ebv[b], semE[b])
        pltpu.async_copy(a_hbm.at[idxs[b]], rowsA[b], semG[b])
        pltpu.async_copy(c_hbm.at[idxd[b]], rowsC[b], semG[b])

    def wait_grp(b):
        pltpu.make_async_copy(eb_hbm.at[pl.ds(0, GRPR), :], ebv[b],
                              semE[b]).wait()
        pltpu.make_async_copy(a_hbm.at[pl.ds(0, GRP), :], rowsA[b],
                              semG[b]).wait()
        pltpu.make_async_copy(a_hbm.at[pl.ds(0, GRP), :], rowsC[b],
                              semG[b]).wait()

    def compute_store(g, b):
        ra, rc, eb_ = rowsA[b], rowsC[b], ebv[b]

        @plsc.parallel_loop(0, GRPR, step=1, unroll=2)
        def _(i):
            for j in range(8):
                sl = pl.ds(j * F, F)
                eb_[i, sl] = jnp.maximum(
                    ra[8 * i + j] + rc[8 * i + j] + eb_[i, sl], 0.0)

        offp = (base_g + g) * GRPR
        pltpu.async_copy(eb_, out_hbm.at[pl.ds(offp, GRPR), :], semO[b])

    def wait_store(b):
        pltpu.make_async_copy(ebv[b], out_hbm.at[pl.ds(0, GRPR), :],
                              semO[b]).wait()

    fire_idx(0, 0)
    wait_idx(0)
    fire_grp(0, 0)
    fire_idx(1, 1)

    def pair(k, _):
        g0 = 2 * k
        wait_idx(1)
        fire_grp(g0 + 1, 1)
        wait_grp(0)
        compute_store(g0, 0)
        wait_grp(1)
        compute_store(g0 + 1, 1)

        @pl.when(k < G // 2 - 1)
        def _():
            wait_store(0)
            fire_idx(g0 + 2, 0)
            wait_idx(0)
            fire_grp(g0 + 2, 0)
            wait_store(1)
            fire_idx(g0 + 3, 1)

        return 0

    lax.fori_loop(0, G // 2, pair, 0)
    wait_store(0)
    wait_store(1)


@functools.lru_cache(maxsize=None)
def _sc_scatter_kernel():
    mesh = plsc.VectorSubcoreMesh(core_axis_name="c", subcore_axis_name="s")
    buf = [pltpu.VMEM((GRP,), _i32),
           pltpu.VMEM((GRP,), _i32),
           pltpu.VMEM((GRP, F), _f32),
           pltpu.VMEM((GRPR, 128), _f32),
           pltpu.SemaphoreType.DMA,
           pltpu.SemaphoreType.DMA,
           pltpu.SemaphoreType.DMA,
           pltpu.SemaphoreType.DMA]
    return functools.partial(
        pl.kernel, mesh=mesh,
        compiler_params=pltpu.CompilerParams(use_tc_tiling_on_sc=False),
        out_type=jax.ShapeDtypeStruct((NCORE, N_TAB, F), _f32),
        scratch_types=buf + buf + [
            pltpu.VMEM((ZB, F), _f32),
            pltpu.VMEM_SHARED((N_TAB, F), _f32),
        ])(_sc_scatter_body)


def _sc_scatter_body(u_hbm, w_hbm, src1_hbm, dst1_hbm, out_hbm, *s):
    idxs = (s[0], s[8])
    idxd = (s[1], s[9])
    rowsU = (s[2], s[10])
    wv = (s[3], s[11])
    semI = (s[4], s[12])
    semG = (s[5], s[13])
    semW = (s[6], s[14])
    semS = (s[7], s[15])
    zbuf = s[16]
    acc = s[17]
    cid = lax.axis_index("c")
    sid = lax.axis_index("s")
    wid = sid * NCORE + cid
    base_g = wid * G

    @plsc.parallel_loop(0, ZB, step=1, unroll=8)
    def _(i):
        zbuf[i] = jnp.zeros((F,), _f32)

    for r in range(PER_SUB // ZB):
        pltpu.sync_copy(zbuf, acc.at[pl.ds(sid * PER_SUB + r * ZB, ZB), :])
    plsc.subcore_barrier()

    def fire_idx(g, b):
        off = (base_g + g) * GRP
        pltpu.async_copy(src1_hbm.at[pl.ds(off, GRP)], idxs[b], semI[b])
        pltpu.async_copy(dst1_hbm.at[pl.ds(off, GRP)], idxd[b], semI[b])

    def wait_idx(b):
        pltpu.make_async_copy(src1_hbm.at[pl.ds(0, GRP)], idxs[b],
                              semI[b]).wait()
        pltpu.make_async_copy(dst1_hbm.at[pl.ds(0, GRP)], idxd[b],
                              semI[b]).wait()

    def fire_grp(g, b):
        offp = (base_g + g) * GRPR
        pltpu.async_copy(w_hbm.at[pl.ds(offp, GRPR), :], wv[b], semW[b])
        pltpu.async_copy(u_hbm.at[idxs[b]], rowsU[b], semG[b])

    def wait_grp(b):
        pltpu.make_async_copy(w_hbm.at[pl.ds(0, GRPR), :], wv[b],
                              semW[b]).wait()
        pltpu.make_async_copy(u_hbm.at[pl.ds(0, GRP), :], rowsU[b],
                              semG[b]).wait()

    def compute_scatter(b):
        ru, w_ = rowsU[b], wv[b]

        @plsc.parallel_loop(0, GRPR, step=1, unroll=2)
        def _(i):
            for j in range(8):
                ru[8 * i + j] = ru[8 * i + j] + w_[i, pl.ds(j * F, F)]

        pltpu.async_copy(ru, acc.at[idxd[b]], semS[b], add=True)

    def wait_scat(b):
        pltpu.make_async_copy(u_hbm.at[pl.ds(0, GRP), :], rowsU[b],
                              semS[b]).wait()

    fire_idx(0, 0)
    wait_idx(0)
    fire_grp(0, 0)
    fire_idx(1, 1)

    def pair(k, _):
        g0 = 2 * k
        wait_idx(1)
        fire_grp(g0 + 1, 1)
        wait_grp(0)
        compute_scatter(0)
        wait_grp(1)
        compute_scatter(1)

        @pl.when(k < G // 2 - 1)
        def _():
            wait_scat(0)
            fire_idx(g0 + 2, 0)
            wait_idx(0)
            fire_grp(g0 + 2, 0)
            wait_scat(1)
            fire_idx(g0 + 3, 1)

        return 0

    lax.fori_loop(0, G // 2, pair, 0)
    wait_scat(0)
    wait_scat(1)
    plsc.subcore_barrier()

    for r in range(PER_SUB // ZB):
        pltpu.sync_copy(acc.at[pl.ds(sid * PER_SUB + r * ZB, ZB), :], zbuf)
        pltpu.sync_copy(zbuf,
                        out_hbm.at[cid, pl.ds(sid * PER_SUB + r * ZB, ZB), :])


def kernel(nte, ete, neW1, neb1, neW2, neb2, eeW1, eeb1, eeW2, eeb2,
           ceW1, ceb1, ceW2, ceb2, cnW1, cnb1, cnW2, cnb2,
           ndW1, ndb1, ndW2, ndb2, ndW3, ndb3,
           edW1, edb1, edW2, edb2, edW3, edb3,
           x_node_types, x_edge_types, edge_index, steps):
    relu = _relu

    def mlp2(x, W1, b1, W2, b2):
        return relu(relu(x @ W1 + b1) @ W2 + b2)

    ntab = mlp2(nte, neW1, neb1, neW2, neb2)
    etab = mlp2(ete, eeW1, eeb1, eeW2, eeb2)
    Wa1, Wa2 = ceW1[0:16], ceW1[16:32]
    Wb1, Wb2 = ceW1[32:48], ceW1[48:64]
    Wc1, Wc2 = ceW1[64:80], ceW1[80:96]
    Wu1, Wu2 = cnW1[0:16], cnW1[16:32]
    M2 = cnW1[32:48]

    def pad8(t):
        return jnp.pad(t, ((0, 8 - t.shape[0]), (0, 0)))

    TN = pad8(jnp.concatenate([ntab @ Wa1, ntab @ Wc1, ntab @ Wu1], axis=1))
    T0 = pad8(jnp.concatenate([ntab @ (Wa1 + Wa2), ntab @ (Wc1 + Wc2),
                               ntab @ (Wu1 + Wu2)], axis=1))
    W3 = jnp.concatenate([Wa2, Wc2, Wu2], axis=1)

    eye8 = jnp.eye(8, dtype=_f32)
    kr = lambda W: jnp.kron(eye8, W)
    t8 = lambda b: jnp.tile(b.reshape(1, -1), (1, 8))
    Tbt = pad8(jnp.tile(etab @ Wb1 + ceb1, (1, 8)))
    Tb0t = pad8(jnp.tile(etab @ (Wb1 + Wb2) + ceb1, (1, 8)))
    ceW2d, M2d, Wb2d = kr(ceW2), kr(M2), kr(Wb2)
    ceb2t = t8(ceb2)
    edW1d, edW2d = kr(edW1), kr(edW2)
    edW3d = kr(jnp.pad(edW3, ((0, 0), (0, F - edW3.shape[1]))))
    edb1t, edb2t = t8(edb1), t8(edb2)
    edb3t = t8(jnp.pad(edb3, (0, F - edb3.shape[0])))

    row = lambda b: b.reshape(1, F)
    cnb1r, cnb2r = row(cnb1), row(cnb2)
    ndb1r, ndb2r = row(ndb1), row(ndb2)
    ndW3p = jnp.pad(ndW3, ((0, 0), (0, 8 - ndW3.shape[1])))
    ndb3p = jnp.pad(ndb3, (0, 8 - ndb3.shape[0])).reshape(1, 8)

    src = edge_index[0].astype(_i32)
    dst = edge_index[1].astype(_i32)
    padE = E_PAD - E
    src1 = jnp.concatenate([src, jnp.full((padE,), DUMMY, _i32)])
    dst1 = jnp.concatenate(
        [dst, DUMMY + (jnp.arange(padE, dtype=_i32) % (N_TAB - N))])
    ntf = jnp.pad(x_node_types.astype(_f32), (0, N_TAB - N)
                  ).reshape(N_TAB, 1)
    etr = jnp.repeat(jnp.pad(x_edge_types.astype(_f32), (0, padE)), F
                     ).reshape(EP8, 128)

    a0, c0, u0 = _tc_init_nodes(ntf, T0)
    eb0 = _tc_init_edges(etr, Tb0t)
    er0 = jnp.zeros((EP8, 128), _f32)
    xn0 = jnp.zeros((N_TAB, F), _f32)

    def body(t, carry):
        a, c, u, eb, er1, xn = carry
        er1 = _sc_gather_kernel()(a, c, eb, src1, dst1)
        w, ebn = _tc_edge(er1, etr, ceW2d, ceb2t, M2d, Wb2d, Tbt)
        S2 = _sc_scatter_kernel()(u, w, src1, dst1)
        xn, a, c, u = _tc_node(S2[0], S2[1], ntf, cnb1r, cnW2, cnb2r, TN, W3)
        return (a, c, u, ebn, er1, xn)

    a, c, u, eb, er1, xn = lax.fori_loop(
        0, steps, body, (a0, c0, u0, eb0, er0, xn0))

    pn = _tc_dec(xn, ndW1, ndb1r, ndW2, ndb2r, ndW3p, ndb3p, N_TAB, BN)
    pep = _tc_dec_edge(er1, ceW2d, ceb2t, edW1d, edb1t, edW2d, edb2t,
                       edW3d, edb3t)
    pe = pep.reshape(E_PAD, F)
    return (pn[:N, :3], pe[:E, :3])
